# Initial kernel scaffold; baseline (speedup 1.0000x reference)
import jax
import jax.numpy as jnp
from jax.experimental import pallas as pl


def kernel(x, table, W, b):
    raise NotImplementedError("write your pallas kernel here")


# speedup vs baseline: 5.0020x; 5.0020x over previous
import functools

import jax
import jax.numpy as jnp
from jax import lax
from jax.experimental import pallas as pl
from jax.experimental.pallas import tpu as pltpu
from jax.experimental.pallas import tpu_sc as plsc

_K = 10
_D = 128
_L16 = 16


def _proj_kernel(table_ref, w_ref, b_ref, m_ref):
    h = jnp.maximum(table_ref[...], 0.0)
    m = lax.dot_general(h, w_ref[...], (((1,), (1,)), ((), ())),
                        preferred_element_type=jnp.float32)
    m_ref[...] = m + b_ref[...]


def _take16(vec, idx):
    dnums = lax.GatherDimensionNumbers(
        offset_dims=(), collapsed_slice_dims=(0,), start_index_map=(0,))
    return lax.gather(vec, idx[:, None], dnums, (1,),
                      mode=lax.GatherScatterMode.PROMISE_IN_BOUNDS)


@functools.lru_cache(maxsize=None)
def _make_sc_lookup(n_total: int, chunk: int):
    info = plsc.get_sparse_core_info()
    num_cores = info.num_cores
    num_workers = info.num_cores * info.num_subcores
    per_worker = n_total // num_workers
    assert per_worker * num_workers == n_total
    num_chunks = per_worker // chunk
    assert num_chunks * chunk == per_worker
    groups = chunk // _L16

    mesh = plsc.VectorSubcoreMesh(core_axis_name="c", subcore_axis_name="s")

    @functools.partial(
        pl.kernel,
        mesh=mesh,
        out_type=jax.ShapeDtypeStruct((n_total * _K,), jnp.float32),
        scratch_types=[
            pltpu.VMEM((128,), jnp.float32),
            pltpu.VMEM((chunk,), jnp.int32),
            pltpu.VMEM((chunk * _K,), jnp.float32),
            pltpu.SemaphoreType.DMA,
        ],
        compiler_params=pltpu.CompilerParams(use_tc_tiling_on_sc=False,
                                             needs_layout_passes=False),
    )
    def sc_lookup(m_hbm, x_hbm, out_hbm, m_v, idx_v, out_c, sem):
        wid = lax.axis_index("s") * num_cores + lax.axis_index("c")
        base = wid * per_worker

        pltpu.sync_copy(m_hbm, m_v)

        iota = lax.iota(jnp.int32, _L16)
        qs, rs = [], []
        for v in range(_K):
            j = iota + (_L16 * v)
            q = lax.shift_right_logical(j * 13108, 17)
            qs.append(q)
            rs.append(j - q * _K)

        def body(c, carry):
            off = base + c * chunk
            pltpu.sync_copy(x_hbm.at[pl.ds(off, chunk)], idx_v)

            def group(g, carry2):
                xv = idx_v[pl.ds(g * _L16, _L16)]
                x10 = xv * _K
                for v in range(_K):
                    midx = _take16(x10, qs[v]) + rs[v]
                    val = plsc.load_gather(m_v, [midx])
                    out_c[pl.ds(g * (_L16 * _K) + v * _L16, _L16)] = val
                return carry2

            lax.fori_loop(0, groups, group, 0)
            pltpu.sync_copy(out_c, out_hbm.at[pl.ds(off * _K, chunk * _K)])
            return carry

        lax.fori_loop(0, num_chunks, body, 0)

    return sc_lookup


def kernel(x, table, W, b):
    B, L = x.shape
    n_total = B * L
    x_flat = x.reshape(n_total).astype(jnp.int32)

    m = pl.pallas_call(
        _proj_kernel,
        out_shape=jax.ShapeDtypeStruct((_K, _K), jnp.float32),
    )(table, W, b.reshape(1, _K))
    m_flat = jnp.zeros((128,), jnp.float32).at[:_K * _K].set(m.reshape(-1))

    out = _make_sc_lookup(n_total, 2048)(m_flat, x_flat)
    return out.reshape(B, L, _K)


# speedup vs baseline: 185.4384x; 37.0732x over previous
import functools

import jax
import jax.numpy as jnp
from jax import lax
from jax.experimental import pallas as pl
from jax.experimental.pallas import tpu as pltpu

_K = 10
_D = 128


def _proj_kernel(table_ref, w_ref, b_ref, m_ref):
    h = jnp.maximum(table_ref[...], 0.0)
    m = lax.dot_general(h, w_ref[...], (((1,), (1,)), ((), ())),
                        preferred_element_type=jnp.float32)
    m_ref[...] = m + b_ref[...]


def _lookup_kernel(m_ref, xt_ref, out_ref):
    x = xt_ref[...]
    masks = [x == i for i in range(1, _K)]
    for k in range(_K):
        val = jnp.broadcast_to(m_ref[0, k], x.shape)
        for i in range(1, _K):
            val = jnp.where(masks[i - 1], m_ref[i, k], val)
        out_ref[k] = val


def kernel(x, table, W, b):
    B, L = x.shape

    m = pl.pallas_call(
        _proj_kernel,
        out_shape=jax.ShapeDtypeStruct((_K, _K), jnp.float32),
    )(table, W, b.reshape(1, _K))

    xt = jnp.transpose(x).astype(jnp.int32)

    bc = 512
    nb = B // bc
    out3 = pl.pallas_call(
        _lookup_kernel,
        grid=(nb,),
        in_specs=[
            pl.BlockSpec(memory_space=pltpu.SMEM),
            pl.BlockSpec((L, bc), lambda i: (0, i)),
        ],
        out_specs=pl.BlockSpec((_K, L, bc), lambda i: (0, 0, i)),
        out_shape=jax.ShapeDtypeStruct((_K, L, B), jnp.float32),
    )(m, xt)

    return jnp.transpose(out3, (2, 1, 0))
